# baseline (speedup 1.0000x reference)
import functools

import jax
import jax.numpy as jnp
from jax import lax
from jax.experimental import pallas as pl
from jax.experimental.pallas import tpu as pltpu
from jax.experimental.pallas import tpu_sc as plsc

_NK = 8
_NP = 80
_SC_CH = 4096


def _round_bf16_bits(x):
    u = jax.lax.bitcast_convert_type(x, jnp.int32)
    odd = jax.lax.shift_right_logical(u, 16) & 1
    u = (u + 0x7FFF + odd) & jnp.int32(-65536)
    return jax.lax.bitcast_convert_type(u, jnp.float32)


def _prep_params(ys, A):
    eps = 0.0001
    neg = jnp.sum(A * (A < 0), axis=1)
    pos = jnp.sum(A * (A > 0), axis=1)
    ys_full = jnp.concatenate([neg[..., None], ys, pos[..., None]], axis=-1)
    lin = jnp.linspace(0.0, 1.0, _NK + 2)
    xs = lin[None, None, :] * (pos + eps - neg)[..., None] + neg[..., None]
    dx0 = xs[..., 1] - xs[..., 0]
    slopes = jnp.diff(ys_full, axis=-1) / dx0[..., None]
    s0 = slopes[..., 0]
    d = slopes[..., 1:] - slopes[..., :-1]
    knots = xs[..., 1:-1]
    c0 = ys_full[..., 0] - s0 * xs[..., 0] - jnp.sum(d * knots, axis=-1)
    pinv = jnp.linalg.pinv(A)
    B = A.shape[0]
    return jnp.concatenate(
        [
            _round_bf16_bits(A.reshape(B, 9)),
            _round_bf16_bits(pinv.reshape(B, 9)),
            s0,
            c0,
            knots.reshape(B, 24),
            d.reshape(B, 24),
            xs[..., 0],
            xs[..., -1],
            jnp.zeros((B, 2), jnp.float32),
        ],
        axis=-1,
    )


def _spline_16px(P, r, g, b):
    r = _round_bf16_bits(r)
    g = _round_bf16_bits(g)
    b = _round_bf16_bits(b)
    ests = []
    for a in range(3):
        x = r * P(a) + g * P(3 + a) + b * P(6 + a)
        f = P(21 + a) + P(18 + a) * x
        for i in range(_NK):
            f = f + P(48 + a * 8 + i) * jnp.maximum(x, P(24 + a * 8 + i))
        oob = (x < P(72 + a)) | (x > P(75 + a))
        f = jnp.where(oob, 99.0, f)
        ests.append(_round_bf16_bits(f))
    return [
        ests[0] * P(9 + c) + ests[1] * P(12 + c) + ests[2] * P(15 + c)
        for c in range(3)
    ]


def _sc_make(B, C, HW):
    half_px = HW // 2
    n_chunks = half_px // _SC_CH
    n_vec = _SC_CH // 16

    @functools.partial(
        pl.kernel,
        out_type=jax.ShapeDtypeStruct((B * C * HW,), jnp.float32),
        mesh=plsc.VectorSubcoreMesh(core_axis_name="c", subcore_axis_name="s"),
        scratch_types=[
            pltpu.VMEM((_NP, 16), jnp.float32),
            pltpu.VMEM((_SC_CH,), jnp.float32),
            pltpu.VMEM((_SC_CH,), jnp.float32),
            pltpu.VMEM((_SC_CH,), jnp.float32),
            pltpu.VMEM((_SC_CH,), jnp.float32),
            pltpu.VMEM((_SC_CH,), jnp.float32),
            pltpu.VMEM((_SC_CH,), jnp.float32),
        ],
    )
    def sck(raw_hbm, par_hbm, out_hbm, par_v, rv, gv, bv, ro, go, bo):
        cidx = lax.axis_index("c")
        b = lax.axis_index("s")
        base = cidx * half_px
        pltpu.sync_copy(par_hbm.at[b], par_v)
        pvec = [par_v[k] for k in range(78)]

        def P(k):
            return pvec[k]

        ins = (rv, gv, bv)
        res = (ro, go, bo)

        def chunk(ci, carry):
            off = base + ci * _SC_CH
            for ch in range(3):
                flat = (b * 3 + ch) * HW + off
                pltpu.sync_copy(raw_hbm.at[pl.ds(flat, _SC_CH)], ins[ch])

            def vec(k2, carry2):
                sl = pl.ds(k2 * 16, 16)
                outs = _spline_16px(P, rv[sl], gv[sl], bv[sl])
                for ch in range(3):
                    res[ch][sl] = outs[ch]
                return carry2

            lax.fori_loop(0, n_vec, vec, 0)
            for ch in range(3):
                flat = (b * 3 + ch) * HW + off
                pltpu.sync_copy(res[ch], out_hbm.at[pl.ds(flat, _SC_CH)])
            return carry

        lax.fori_loop(0, n_chunks, chunk, 0)

    return sck


@jax.jit
def kernel(raw, ys, A):
    B, C, H, W = raw.shape
    HW = H * W
    params = _prep_params(ys, A)
    par_bcast = jnp.broadcast_to(params[:, :, None], (B, _NP, 16))
    rview = raw.reshape(B * C * HW)
    out = _sc_make(B, C, HW)(rview, par_bcast)
    return out.reshape(B, C, H, W)


# speedup vs baseline: 2.4882x; 2.4882x over previous
import functools

import jax
import jax.numpy as jnp
from jax import lax
from jax.experimental import pallas as pl
from jax.experimental.pallas import tpu as pltpu
from jax.experimental.pallas import tpu_sc as plsc

_NK = 8
_NP = 80
_SC_CH = 4096


def _round_bf16_bits(x):
    u = jax.lax.bitcast_convert_type(x, jnp.int32)
    odd = jax.lax.shift_right_logical(u, 16) & 1
    u = (u + 0x7FFF + odd) & jnp.int32(-65536)
    return jax.lax.bitcast_convert_type(u, jnp.float32)


def _prep_params(ys, A):
    eps = 0.0001
    neg = jnp.sum(A * (A < 0), axis=1)
    pos = jnp.sum(A * (A > 0), axis=1)
    ys_full = jnp.concatenate([neg[..., None], ys, pos[..., None]], axis=-1)
    lin = jnp.linspace(0.0, 1.0, _NK + 2)
    xs = lin[None, None, :] * (pos + eps - neg)[..., None] + neg[..., None]
    dx0 = xs[..., 1] - xs[..., 0]
    slopes = jnp.diff(ys_full, axis=-1) / dx0[..., None]
    s0 = slopes[..., 0]
    d = slopes[..., 1:] - slopes[..., :-1]
    knots = xs[..., 1:-1]
    c0 = ys_full[..., 0] - s0 * xs[..., 0] - jnp.sum(d * knots, axis=-1)
    pinv = jnp.linalg.pinv(A)
    B = A.shape[0]
    return jnp.concatenate(
        [
            _round_bf16_bits(A.reshape(B, 9)),
            _round_bf16_bits(pinv.reshape(B, 9)),
            s0,
            c0,
            knots.reshape(B, 24),
            d.reshape(B, 24),
            xs[..., 0],
            xs[..., -1],
            jnp.zeros((B, 2), jnp.float32),
        ],
        axis=-1,
    )


def _spline_16px(P, r, g, b):
    r = _round_bf16_bits(r)
    g = _round_bf16_bits(g)
    b = _round_bf16_bits(b)
    ests = []
    for a in range(3):
        x = r * P(a) + g * P(3 + a) + b * P(6 + a)
        f = P(21 + a) + P(18 + a) * x
        for i in range(_NK):
            f = f + P(48 + a * 8 + i) * jnp.maximum(x, P(24 + a * 8 + i))
        oob = (x < P(72 + a)) | (x > P(75 + a))
        f = jnp.where(oob, 99.0, f)
        ests.append(_round_bf16_bits(f))
    return [
        ests[0] * P(9 + c) + ests[1] * P(12 + c) + ests[2] * P(15 + c)
        for c in range(3)
    ]


def _tc_body(params_ref, raw_ref, out_ref):
    def P(k):
        return params_ref[0, 0, k]

    outs = _spline_16px(P, raw_ref[0, 0], raw_ref[0, 1], raw_ref[0, 2])
    for c in range(3):
        out_ref[0, c] = outs[c]


def _sc_make(B, C, H, W, H_tc, H_sc):
    half_px = H_sc * W // 2
    n_chunks = half_px // _SC_CH
    n_vec = _SC_CH // 16

    @functools.partial(
        pl.kernel,
        out_type=jax.ShapeDtypeStruct((B * C * H_sc * W,), jnp.float32),
        mesh=plsc.VectorSubcoreMesh(core_axis_name="c", subcore_axis_name="s"),
        scratch_types=[
            pltpu.VMEM((_NP, 16), jnp.float32),
            pltpu.VMEM((_SC_CH,), jnp.float32),
            pltpu.VMEM((_SC_CH,), jnp.float32),
            pltpu.VMEM((_SC_CH,), jnp.float32),
            pltpu.VMEM((_SC_CH,), jnp.float32),
            pltpu.VMEM((_SC_CH,), jnp.float32),
            pltpu.VMEM((_SC_CH,), jnp.float32),
        ],
    )
    def sck(raw_hbm, par_hbm, out_hbm, par_v, rv, gv, bv, ro, go, bo):
        cidx = lax.axis_index("c")
        b = lax.axis_index("s")
        pltpu.sync_copy(par_hbm.at[b], par_v)
        pvec = [par_v[k] for k in range(78)]

        def P(k):
            return pvec[k]

        ins = (rv, gv, bv)
        res = (ro, go, bo)

        def chunk(ci, carry):
            off = ci * _SC_CH
            for ch in range(3):
                src = ((b * 3 + ch) * H + H_tc) * W + cidx * half_px + off
                pltpu.sync_copy(raw_hbm.at[pl.ds(src, _SC_CH)], ins[ch])

            def vec(k2, carry2):
                sl = pl.ds(k2 * 16, 16)
                outs = _spline_16px(P, rv[sl], gv[sl], bv[sl])
                for ch in range(3):
                    res[ch][sl] = outs[ch]
                return carry2

            lax.fori_loop(0, n_vec, vec, 0)
            for ch in range(3):
                dst = (b * 3 + ch) * H_sc * W + cidx * half_px + off
                pltpu.sync_copy(res[ch], out_hbm.at[pl.ds(dst, _SC_CH)])
            return carry

        lax.fori_loop(0, n_chunks, chunk, 0)

    return sck


_H_TC = 400
_HB_TC = 200


@jax.jit
def kernel(raw, ys, A):
    B, C, H, W = raw.shape
    H_sc = H - _H_TC
    params = _prep_params(ys, A)
    par_sm = params.reshape(B, 1, _NP)
    par_bcast = jnp.broadcast_to(params[:, :, None], (B, _NP, 16))
    rflat = raw.reshape(B * C * H * W)
    sc_out = _sc_make(B, C, H, W, _H_TC, H_sc)(rflat, par_bcast)
    tc_out = pl.pallas_call(
        _tc_body,
        grid=(B, _H_TC // _HB_TC),
        in_specs=[
            pl.BlockSpec(
                (1, 1, _NP), lambda b, h: (b, 0, 0), memory_space=pltpu.SMEM
            ),
            pl.BlockSpec((1, C, _HB_TC, W), lambda b, h: (b, 0, h, 0)),
        ],
        out_specs=pl.BlockSpec((1, C, _HB_TC, W), lambda b, h: (b, 0, h, 0)),
        out_shape=jax.ShapeDtypeStruct((B, C, _H_TC, W), raw.dtype),
    )(par_sm, raw)
    return jnp.concatenate(
        [tc_out, sc_out.reshape(B, C, H_sc, W)], axis=2
    )


# speedup vs baseline: 3.8946x; 1.5652x over previous
import jax
import jax.numpy as jnp
from jax.experimental import pallas as pl
from jax.experimental.pallas import tpu as pltpu

_NK = 8
_NP = 80


def _round_bf16_bits(x):
    u = jax.lax.bitcast_convert_type(x, jnp.int32)
    odd = jax.lax.shift_right_logical(u, 16) & 1
    u = (u + 0x7FFF + odd) & jnp.int32(-65536)
    return jax.lax.bitcast_convert_type(u, jnp.float32)


def _prep_params(ys, A):
    eps = 0.0001
    neg = jnp.sum(A * (A < 0), axis=1)
    pos = jnp.sum(A * (A > 0), axis=1)
    ys_full = jnp.concatenate([neg[..., None], ys, pos[..., None]], axis=-1)
    lin = jnp.linspace(0.0, 1.0, _NK + 2)
    xs = lin[None, None, :] * (pos + eps - neg)[..., None] + neg[..., None]
    dx0 = xs[..., 1] - xs[..., 0]
    slopes = jnp.diff(ys_full, axis=-1) / dx0[..., None]
    s0 = slopes[..., 0]
    d = slopes[..., 1:] - slopes[..., :-1]
    knots = xs[..., 1:-1]
    c0 = ys_full[..., 0] - s0 * xs[..., 0] - jnp.sum(d * knots, axis=-1)
    pinv = jnp.linalg.pinv(A)
    B = A.shape[0]
    return jnp.concatenate(
        [
            _round_bf16_bits(A.reshape(B, 9)),
            _round_bf16_bits(pinv.reshape(B, 9)),
            s0,
            c0,
            knots.reshape(B, 24),
            d.reshape(B, 24),
            xs[..., 0],
            xs[..., -1],
            jnp.zeros((B, 2), jnp.float32),
        ],
        axis=-1,
    ).reshape(B, 1, _NP)


def _round_bf16_split(v):
    t = v * jnp.float32(65537.0)
    return t - (t - v)


def _tc_body(params_ref, raw_ref, out_ref):
    r = _round_bf16_split(raw_ref[0, 0])
    g = _round_bf16_split(raw_ref[0, 1])
    b = _round_bf16_split(raw_ref[0, 2])

    def P(k):
        return params_ref[0, 0, k]

    ests = []
    for a in range(3):
        x = r * P(a) + g * P(3 + a) + b * P(6 + a)
        f = P(21 + a) + P(18 + a) * x
        for i in range(_NK):
            f = f + P(48 + a * 8 + i) * jnp.maximum(x, P(24 + a * 8 + i))
        oob = (x < P(72 + a)) | (x > P(75 + a))
        f = jnp.where(oob, 99.0, f)
        ests.append(_round_bf16_split(f))
    for c in range(3):
        out_ref[0, c] = (
            ests[0] * P(9 + c) + ests[1] * P(12 + c) + ests[2] * P(15 + c)
        )


@jax.jit
def kernel(raw, ys, A):
    B, C, H, W = raw.shape
    params = _prep_params(ys, A)
    HB = 256
    out = pl.pallas_call(
        _tc_body,
        grid=(B, H // HB),
        in_specs=[
            pl.BlockSpec(
                (1, 1, _NP), lambda b, h: (b, 0, 0), memory_space=pltpu.SMEM
            ),
            pl.BlockSpec((1, C, HB, W), lambda b, h: (b, 0, h, 0)),
        ],
        out_specs=pl.BlockSpec((1, C, HB, W), lambda b, h: (b, 0, h, 0)),
        out_shape=jax.ShapeDtypeStruct(raw.shape, raw.dtype),
    )(params, raw)
    return out


# speedup vs baseline: 3.8972x; 1.0007x over previous
import jax
import jax.numpy as jnp
from jax.experimental import pallas as pl
from jax.experimental.pallas import tpu as pltpu

_NK = 8
_NP = 80


def _round_bf16_bits(x):
    u = jax.lax.bitcast_convert_type(x, jnp.int32)
    odd = jax.lax.shift_right_logical(u, 16) & 1
    u = (u + 0x7FFF + odd) & jnp.int32(-65536)
    return jax.lax.bitcast_convert_type(u, jnp.float32)


def _prep_params(ys, A):
    eps = 0.0001
    neg = jnp.sum(A * (A < 0), axis=1)
    pos = jnp.sum(A * (A > 0), axis=1)
    ys_full = jnp.concatenate([neg[..., None], ys, pos[..., None]], axis=-1)
    lin = jnp.linspace(0.0, 1.0, _NK + 2)
    xs = lin[None, None, :] * (pos + eps - neg)[..., None] + neg[..., None]
    dx0 = xs[..., 1] - xs[..., 0]
    slopes = jnp.diff(ys_full, axis=-1) / dx0[..., None]
    s0 = slopes[..., 0]
    d = slopes[..., 1:] - slopes[..., :-1]
    knots = xs[..., 1:-1]
    c0 = ys_full[..., 0] - s0 * xs[..., 0] - jnp.sum(d * knots, axis=-1)
    pinv = jnp.linalg.pinv(A)
    B = A.shape[0]
    return jnp.concatenate(
        [
            _round_bf16_bits(A.reshape(B, 9)),
            _round_bf16_bits(pinv.reshape(B, 9)),
            s0,
            c0,
            knots.reshape(B, 24),
            d.reshape(B, 24),
            xs[..., 0],
            xs[..., -1],
            jnp.zeros((B, 2), jnp.float32),
        ],
        axis=-1,
    ).reshape(B, 1, _NP)


def _round_bf16_split(v):
    t = v * jnp.float32(65537.0)
    return t - (t - v)


def _tc_body(params_ref, raw_ref, out_ref):
    r = _round_bf16_split(raw_ref[0, 0])
    g = _round_bf16_split(raw_ref[0, 1])
    b = _round_bf16_split(raw_ref[0, 2])

    def P(k):
        return params_ref[0, 0, k]

    ests = []
    for a in range(3):
        x = r * P(a) + g * P(3 + a) + b * P(6 + a)
        f = P(21 + a) + P(18 + a) * x
        for i in range(_NK):
            f = f + P(48 + a * 8 + i) * jnp.maximum(x, P(24 + a * 8 + i))
        oob = (x < P(72 + a)) | (x > P(75 + a))
        f = jnp.where(oob, 99.0, f)
        ests.append(_round_bf16_split(f))
    for c in range(3):
        out_ref[0, c] = (
            ests[0] * P(9 + c) + ests[1] * P(12 + c) + ests[2] * P(15 + c)
        )


@jax.jit
def kernel(raw, ys, A):
    B, C, H, W = raw.shape
    params = _prep_params(ys, A)
    HB = 512
    out = pl.pallas_call(
        _tc_body,
        grid=(B, H // HB),
        in_specs=[
            pl.BlockSpec(
                (1, 1, _NP), lambda b, h: (b, 0, 0), memory_space=pltpu.SMEM
            ),
            pl.BlockSpec((1, C, HB, W), lambda b, h: (b, 0, h, 0)),
        ],
        out_specs=pl.BlockSpec((1, C, HB, W), lambda b, h: (b, 0, h, 0)),
        out_shape=jax.ShapeDtypeStruct(raw.shape, raw.dtype),
    )(params, raw)
    return out
